# initial kernel scaffold (unmeasured)
import jax
import jax.numpy as jnp
from jax import lax
from jax.experimental import pallas as pl
from jax.experimental.pallas import tpu as pltpu


def kernel(
    x,
):
    def body(*refs):
        pass

    out_shape = jax.ShapeDtypeStruct(..., jnp.float32)
    return pl.pallas_call(body, out_shape=out_shape)(...)


# baseline (device time: 30140 ns/iter reference)
import jax
import jax.numpy as jnp
from jax import lax
from jax.experimental import pallas as pl
from jax.experimental.pallas import tpu as pltpu


def kernel(x):
    _, m, n = x.shape
    half = n // 2

    def body(x_ref, out_ref, comm_ref, send_sem, recv_sem):
        my_x = lax.axis_index("x")
        my_y = lax.axis_index("y")
        my_z = lax.axis_index("z")
        other_x = 1 - my_x

        barrier_sem = pltpu.get_barrier_semaphore()
        pl.semaphore_signal(
            barrier_sem,
            inc=1,
            device_id=(other_x, my_y, my_z),
            device_id_type=pl.DeviceIdType.MESH,
        )
        pl.semaphore_wait(barrier_sem, 1)

        rdma = pltpu.make_async_remote_copy(
            src_ref=x_ref.at[0, :, pl.ds(other_x * half, half)],
            dst_ref=comm_ref,
            send_sem=send_sem,
            recv_sem=recv_sem,
            device_id=(other_x, my_y, my_z),
            device_id_type=pl.DeviceIdType.MESH,
        )
        rdma.start()
        rdma.wait()

        out_ref[:, :] = x_ref[0, :, pl.ds(my_x * half, half)] + comm_ref[:, :]

    return pl.pallas_call(
        body,
        out_shape=jax.ShapeDtypeStruct((m, half), jnp.float32),
        in_specs=[pl.BlockSpec(memory_space=pltpu.VMEM)],
        out_specs=pl.BlockSpec(memory_space=pltpu.VMEM),
        scratch_shapes=[
            pltpu.VMEM((m, half), jnp.float32),
            pltpu.SemaphoreType.DMA,
            pltpu.SemaphoreType.DMA,
        ],
        compiler_params=pltpu.CompilerParams(collective_id=0),
    )(x)


# device time: 26216 ns/iter; 1.1497x vs baseline; 1.1497x over previous
import jax
import jax.numpy as jnp
from jax import lax
from jax.experimental import pallas as pl
from jax.experimental.pallas import tpu as pltpu


def kernel(x):
    _, m, n = x.shape
    half = n // 2
    qm = m // 4
    hq = qm // 2

    def body(
        x_ref,
        out_ref,
        p_ref,
        sem_sx, sem_rx,
        sem_say, sem_ray,
        sem_saz, sem_raz,
        sem_sby, sem_rby,
        sem_sbz, sem_rbz,
    ):
        my_x = lax.axis_index("x")
        my_y = lax.axis_index("y")
        my_z = lax.axis_index("z")
        x_nbr = (1 - my_x, my_y, my_z)
        y_nbr = (my_x, 1 - my_y, my_z)
        z_nbr = (my_x, my_y, 1 - my_z)

        q = 2 * my_y + my_z
        qy = 2 * (1 - my_y) + my_z
        qz = 2 * my_y + (1 - my_z)
        qd = 2 * (1 - my_y) + (1 - my_z)

        barrier_sem = pltpu.get_barrier_semaphore()
        for nbr in (x_nbr, y_nbr, z_nbr):
            pl.semaphore_signal(
                barrier_sem, inc=1, device_id=nbr,
                device_id_type=pl.DeviceIdType.MESH,
            )
        pl.semaphore_wait(barrier_sem, 3)

        rdma_x = pltpu.make_async_remote_copy(
            src_ref=x_ref.at[0, pl.ds(q * qm, qm), pl.ds((1 - my_x) * half, half)],
            dst_ref=p_ref,
            send_sem=sem_sx,
            recv_sem=sem_rx,
            device_id=x_nbr,
            device_id_type=pl.DeviceIdType.MESH,
        )
        rdma_x.start()
        rdma_x.wait()

        out_ref[pl.ds(q * qm, qm), :] = (
            x_ref[0, pl.ds(q * qm, qm), pl.ds(my_x * half, half)] + p_ref[:, :]
        )

        rdma_ay = pltpu.make_async_remote_copy(
            src_ref=out_ref.at[pl.ds(q * qm, qm), :],
            dst_ref=out_ref.at[pl.ds(q * qm, qm), :],
            send_sem=sem_say,
            recv_sem=sem_ray,
            device_id=y_nbr,
            device_id_type=pl.DeviceIdType.MESH,
        )
        rdma_az = pltpu.make_async_remote_copy(
            src_ref=out_ref.at[pl.ds(q * qm, qm), :],
            dst_ref=out_ref.at[pl.ds(q * qm, qm), :],
            send_sem=sem_saz,
            recv_sem=sem_raz,
            device_id=z_nbr,
            device_id_type=pl.DeviceIdType.MESH,
        )
        rdma_ay.start()
        rdma_az.start()
        rdma_ay.wait()
        rdma_az.wait()

        rdma_by = pltpu.make_async_remote_copy(
            src_ref=out_ref.at[pl.ds(qz * qm, hq), :],
            dst_ref=out_ref.at[pl.ds(qz * qm, hq), :],
            send_sem=sem_sby,
            recv_sem=sem_rby,
            device_id=y_nbr,
            device_id_type=pl.DeviceIdType.MESH,
        )
        rdma_bz = pltpu.make_async_remote_copy(
            src_ref=out_ref.at[pl.ds(qy * qm + hq, hq), :],
            dst_ref=out_ref.at[pl.ds(qy * qm + hq, hq), :],
            send_sem=sem_sbz,
            recv_sem=sem_rbz,
            device_id=z_nbr,
            device_id_type=pl.DeviceIdType.MESH,
        )
        rdma_by.start()
        rdma_bz.start()
        rdma_by.wait()
        rdma_bz.wait()

    return pl.pallas_call(
        body,
        out_shape=jax.ShapeDtypeStruct((m, half), jnp.float32),
        in_specs=[pl.BlockSpec(memory_space=pltpu.VMEM)],
        out_specs=pl.BlockSpec(memory_space=pltpu.VMEM),
        scratch_shapes=[
            pltpu.VMEM((qm, half), jnp.float32),
        ]
        + [pltpu.SemaphoreType.DMA] * 10,
        compiler_params=pltpu.CompilerParams(collective_id=0),
    )(x)


# device time: 23494 ns/iter; 1.2829x vs baseline; 1.1159x over previous
import jax
import jax.numpy as jnp
from jax import lax
from jax.experimental import pallas as pl
from jax.experimental.pallas import tpu as pltpu


def kernel(x):
    _, m, n = x.shape
    half = n // 2
    qm = m // 4
    hq = qm // 2

    def body(x_ref, out_ref, p_ref, ssem, rsem):
        my_x = lax.axis_index("x")
        my_y = lax.axis_index("y")
        my_z = lax.axis_index("z")
        x_nbr = (1 - my_x, my_y, my_z)
        y_nbr = (my_x, 1 - my_y, my_z)
        z_nbr = (my_x, my_y, 1 - my_z)

        q = 2 * my_y + my_z
        qy = 2 * (1 - my_y) + my_z
        qz = 2 * my_y + (1 - my_z)

        barrier_sem = pltpu.get_barrier_semaphore()
        for nbr in (x_nbr, y_nbr, z_nbr):
            pl.semaphore_signal(
                barrier_sem, inc=1, device_id=nbr,
                device_id_type=pl.DeviceIdType.MESH,
            )
        pl.semaphore_wait(barrier_sem, 3)

        def rcopy(slot, src, dst, dev):
            return pltpu.make_async_remote_copy(
                src_ref=src, dst_ref=dst,
                send_sem=ssem.at[slot], recv_sem=rsem.at[slot],
                device_id=dev, device_id_type=pl.DeviceIdType.MESH,
            )

        pcols = pl.ds((1 - my_x) * half, half)
        xr = [
            rcopy(c, x_ref.at[0, pl.ds(q * qm + c * hq, hq), pcols],
                  p_ref.at[c], x_nbr)
            for c in range(2)
        ]
        xr[0].start()
        xr[1].start()

        def a_copies(c):
            rows = pl.ds(q * qm + c * hq, hq)
            ay = rcopy(2 + c, out_ref.at[rows, :], out_ref.at[rows, :], y_nbr)
            az = rcopy(4 + c, out_ref.at[rows, :], out_ref.at[rows, :], z_nbr)
            return ay, az

        a_rdmas = []
        for c in range(2):
            xr[c].wait_recv()
            rows = pl.ds(q * qm + c * hq, hq)
            out_ref[rows, :] = (
                x_ref[0, pl.ds(q * qm + c * hq, hq), pl.ds(my_x * half, half)]
                + p_ref[c, :, :]
            )
            ay, az = a_copies(c)
            ay.start()
            az.start()
            a_rdmas.append((ay, az))

        a_rdmas[0][1].wait_recv()
        by = rcopy(6, out_ref.at[pl.ds(qz * qm, hq), :],
                   out_ref.at[pl.ds(qz * qm, hq), :], y_nbr)
        by.start()

        a_rdmas[1][0].wait_recv()
        bz = rcopy(7, out_ref.at[pl.ds(qy * qm + hq, hq), :],
                   out_ref.at[pl.ds(qy * qm + hq, hq), :], z_nbr)
        bz.start()

        a_rdmas[0][0].wait_recv()
        a_rdmas[1][1].wait_recv()
        by.wait_recv()
        bz.wait_recv()
        for r in xr:
            r.wait_send()
        for ay, az in a_rdmas:
            ay.wait_send()
            az.wait_send()
        by.wait_send()
        bz.wait_send()

    return pl.pallas_call(
        body,
        out_shape=jax.ShapeDtypeStruct((m, half), jnp.float32),
        in_specs=[pl.BlockSpec(memory_space=pltpu.VMEM)],
        out_specs=pl.BlockSpec(memory_space=pltpu.VMEM),
        scratch_shapes=[
            pltpu.VMEM((2, hq, half), jnp.float32),
            pltpu.SemaphoreType.DMA((8,)),
            pltpu.SemaphoreType.DMA((8,)),
        ],
        compiler_params=pltpu.CompilerParams(collective_id=0),
    )(x)


# device time: 20939 ns/iter; 1.4394x vs baseline; 1.1220x over previous
import jax
import jax.numpy as jnp
from jax import lax
from jax.experimental import pallas as pl
from jax.experimental.pallas import tpu as pltpu

K = 4


def kernel(x):
    _, m, n = x.shape
    half = n // 2
    qm = m // 4
    cm = qm // K

    def body(x_ref, out_ref, p_ref, ssem, rsem):
        my_x = lax.axis_index("x")
        my_y = lax.axis_index("y")
        my_z = lax.axis_index("z")
        x_nbr = (1 - my_x, my_y, my_z)
        y_nbr = (my_x, 1 - my_y, my_z)
        z_nbr = (my_x, my_y, 1 - my_z)

        q = 2 * my_y + my_z
        qy = 2 * (1 - my_y) + my_z
        qz = 2 * my_y + (1 - my_z)

        barrier_sem = pltpu.get_barrier_semaphore()
        for nbr in (x_nbr, y_nbr, z_nbr):
            pl.semaphore_signal(
                barrier_sem, inc=1, device_id=nbr,
                device_id_type=pl.DeviceIdType.MESH,
            )
        pl.semaphore_wait(barrier_sem, 3)

        def rcopy(slot, src, dst, dev):
            return pltpu.make_async_remote_copy(
                src_ref=src, dst_ref=dst,
                send_sem=ssem.at[slot], recv_sem=rsem.at[slot],
                device_id=dev, device_id_type=pl.DeviceIdType.MESH,
            )

        pcols = pl.ds((1 - my_x) * half, half)
        xr = [
            rcopy(c, x_ref.at[0, pl.ds(q * qm + c * cm, cm), pcols],
                  p_ref.at[c], x_nbr)
            for c in range(K)
        ]
        for c in range(K):
            xr[c].start()

        a_rdmas = []
        for c in range(K):
            xr[c].wait_recv()
            rows = pl.ds(q * qm + c * cm, cm)
            out_ref[rows, :] = (
                x_ref[0, pl.ds(q * qm + c * cm, cm), pl.ds(my_x * half, half)]
                + p_ref[c, :, :]
            )
            ay = rcopy(4 + c, out_ref.at[rows, :], out_ref.at[rows, :], y_nbr)
            az = rcopy(8 + c, out_ref.at[rows, :], out_ref.at[rows, :], z_nbr)
            ay.start()
            az.start()
            a_rdmas.append((ay, az))

        b_rdmas = []
        for c in range(K // 2):
            a_rdmas[c][1].wait_recv()
            rows = pl.ds(qz * qm + c * cm, cm)
            by = rcopy(12 + c, out_ref.at[rows, :], out_ref.at[rows, :], y_nbr)
            by.start()
            b_rdmas.append(by)
        for c in range(K // 2, K):
            a_rdmas[c][0].wait_recv()
            rows = pl.ds(qy * qm + c * cm, cm)
            bz = rcopy(12 + c, out_ref.at[rows, :], out_ref.at[rows, :], z_nbr)
            bz.start()
            b_rdmas.append(bz)

        for c in range(K // 2):
            a_rdmas[c][0].wait_recv()
            a_rdmas[K // 2 + c][1].wait_recv()
        for b in b_rdmas:
            b.wait_recv()
        for r in xr:
            r.wait_send()
        for ay, az in a_rdmas:
            ay.wait_send()
            az.wait_send()
        for b in b_rdmas:
            b.wait_send()

    return pl.pallas_call(
        body,
        out_shape=jax.ShapeDtypeStruct((m, half), jnp.float32),
        in_specs=[pl.BlockSpec(memory_space=pltpu.VMEM)],
        out_specs=pl.BlockSpec(memory_space=pltpu.VMEM),
        scratch_shapes=[
            pltpu.VMEM((K, cm, half), jnp.float32),
            pltpu.SemaphoreType.DMA((16,)),
            pltpu.SemaphoreType.DMA((16,)),
        ],
        compiler_params=pltpu.CompilerParams(collective_id=0),
    )(x)


# device time: 20791 ns/iter; 1.4497x vs baseline; 1.0071x over previous
import jax
import jax.numpy as jnp
from jax import lax
from jax.experimental import pallas as pl
from jax.experimental.pallas import tpu as pltpu

K = 4


def kernel(x):
    _, m, n = x.shape
    half = n // 2
    qm = m // 4
    cm = qm // K

    def body(x_ref, out_ref, p_ref, ssem, rsem):
        my_x = lax.axis_index("x")
        my_y = lax.axis_index("y")
        my_z = lax.axis_index("z")
        x_nbr = (1 - my_x, my_y, my_z)
        y_nbr = (my_x, 1 - my_y, my_z)
        z_nbr = (my_x, my_y, 1 - my_z)

        q = 2 * my_y + my_z
        qy = 2 * (1 - my_y) + my_z
        qz = 2 * my_y + (1 - my_z)
        qd = 2 * (1 - my_y) + (1 - my_z)

        mycols = pl.ds(my_x * half, half)

        barrier_sem = pltpu.get_barrier_semaphore()
        for nbr in (x_nbr, y_nbr, z_nbr):
            pl.semaphore_signal(
                barrier_sem, inc=1, device_id=nbr,
                device_id_type=pl.DeviceIdType.MESH,
            )
        pl.semaphore_wait(barrier_sem, 3)

        def rcopy(slot, src, dst, dev):
            return pltpu.make_async_remote_copy(
                src_ref=src, dst_ref=dst,
                send_sem=ssem.at[slot], recv_sem=rsem.at[slot],
                device_id=dev, device_id_type=pl.DeviceIdType.MESH,
            )

        pcols = pl.ds((1 - my_x) * half, half)
        xr = [
            rcopy(c, x_ref.at[0, pl.ds(q * qm + c * cm, cm), pcols],
                  p_ref.at[c], x_nbr)
            for c in range(K)
        ] + [
            rcopy(K + c, x_ref.at[0, pl.ds(qd * qm + c * cm, cm), pcols],
                  p_ref.at[K + c], x_nbr)
            for c in range(2)
        ]
        for r in xr:
            r.start()

        a_rdmas = []
        for c in range(K):
            xr[c].wait_recv()
            rows = pl.ds(q * qm + c * cm, cm)
            out_ref[rows, :] = x_ref[0, rows, mycols] + p_ref[c, :, :]
            ay = rcopy(6 + c, out_ref.at[rows, :], out_ref.at[rows, :], y_nbr)
            az = rcopy(10 + c, out_ref.at[rows, :], out_ref.at[rows, :], z_nbr)
            ay.start()
            az.start()
            a_rdmas.append((ay, az))

        a_rdmas[2][1].wait_recv()
        rows = pl.ds(qz * qm + 2 * cm, cm)
        by = rcopy(14, out_ref.at[rows, :], out_ref.at[rows, :], y_nbr)
        by.start()

        a_rdmas[3][0].wait_recv()
        rows = pl.ds(qy * qm + 3 * cm, cm)
        bz = rcopy(15, out_ref.at[rows, :], out_ref.at[rows, :], z_nbr)
        bz.start()

        for c in range(2):
            xr[K + c].wait_recv()
            rows = pl.ds(qd * qm + c * cm, cm)
            out_ref[rows, :] = x_ref[0, rows, mycols] + p_ref[K + c, :, :]

        for c in (0, 1, 2):
            a_rdmas[c][0].wait_recv()
        for c in (0, 1, 3):
            a_rdmas[c][1].wait_recv()
        by.wait_recv()
        bz.wait_recv()
        for r in xr:
            r.wait_send()
        for ay, az in a_rdmas:
            ay.wait_send()
            az.wait_send()
        by.wait_send()
        bz.wait_send()

    return pl.pallas_call(
        body,
        out_shape=jax.ShapeDtypeStruct((m, half), jnp.float32),
        in_specs=[pl.BlockSpec(memory_space=pltpu.VMEM)],
        out_specs=pl.BlockSpec(memory_space=pltpu.VMEM),
        scratch_shapes=[
            pltpu.VMEM((K + 2, cm, half), jnp.float32),
            pltpu.SemaphoreType.DMA((16,)),
            pltpu.SemaphoreType.DMA((16,)),
        ],
        compiler_params=pltpu.CompilerParams(collective_id=0),
    )(x)


# device time: 19397 ns/iter; 1.5538x vs baseline; 1.0719x over previous
import jax
import jax.numpy as jnp
from jax import lax
from jax.experimental import pallas as pl
from jax.experimental.pallas import tpu as pltpu

K = 8


def kernel(x):
    _, m, n = x.shape
    half = n // 2
    qm = m // 4
    cm = qm // K

    n_x = K + K // 2
    ay0, az0, b0 = n_x, n_x + K, n_x + 2 * K
    n_sem = b0 + K // 2

    def body(x_ref, out_ref, p_ref, ssem, rsem):
        my_x = lax.axis_index("x")
        my_y = lax.axis_index("y")
        my_z = lax.axis_index("z")
        x_nbr = (1 - my_x, my_y, my_z)
        y_nbr = (my_x, 1 - my_y, my_z)
        z_nbr = (my_x, my_y, 1 - my_z)

        q = 2 * my_y + my_z
        qy = 2 * (1 - my_y) + my_z
        qz = 2 * my_y + (1 - my_z)
        qd = 2 * (1 - my_y) + (1 - my_z)

        mycols = pl.ds(my_x * half, half)

        barrier_sem = pltpu.get_barrier_semaphore()
        for nbr in (x_nbr, y_nbr, z_nbr):
            pl.semaphore_signal(
                barrier_sem, inc=1, device_id=nbr,
                device_id_type=pl.DeviceIdType.MESH,
            )
        pl.semaphore_wait(barrier_sem, 3)

        def rcopy(slot, src, dst, dev):
            return pltpu.make_async_remote_copy(
                src_ref=src, dst_ref=dst,
                send_sem=ssem.at[slot], recv_sem=rsem.at[slot],
                device_id=dev, device_id_type=pl.DeviceIdType.MESH,
            )

        pcols = pl.ds((1 - my_x) * half, half)
        xr = [
            rcopy(c, x_ref.at[0, pl.ds(q * qm + c * cm, cm), pcols],
                  p_ref.at[c], x_nbr)
            for c in range(K)
        ] + [
            rcopy(K + c, x_ref.at[0, pl.ds(qd * qm + c * cm, cm), pcols],
                  p_ref.at[K + c], x_nbr)
            for c in range(K // 2)
        ]
        for r in xr:
            r.start()

        a_rdmas = []
        for c in range(K):
            xr[c].wait_recv()
            rows = pl.ds(q * qm + c * cm, cm)
            out_ref[rows, :] = x_ref[0, rows, mycols] + p_ref[c, :, :]
            ay = rcopy(ay0 + c, out_ref.at[rows, :], out_ref.at[rows, :], y_nbr)
            az = rcopy(az0 + c, out_ref.at[rows, :], out_ref.at[rows, :], z_nbr)
            ay.start()
            az.start()
            a_rdmas.append((ay, az))

        b_rdmas = []
        for i, c in enumerate(range(K // 2, 3 * K // 4)):
            a_rdmas[c][1].wait_recv()
            rows = pl.ds(qz * qm + c * cm, cm)
            by = rcopy(b0 + i, out_ref.at[rows, :], out_ref.at[rows, :], y_nbr)
            by.start()
            b_rdmas.append(by)
        for i, c in enumerate(range(3 * K // 4, K)):
            a_rdmas[c][0].wait_recv()
            rows = pl.ds(qy * qm + c * cm, cm)
            bz = rcopy(b0 + K // 4 + i, out_ref.at[rows, :],
                       out_ref.at[rows, :], z_nbr)
            bz.start()
            b_rdmas.append(bz)

        for c in range(K // 2):
            xr[K + c].wait_recv()
            rows = pl.ds(qd * qm + c * cm, cm)
            out_ref[rows, :] = x_ref[0, rows, mycols] + p_ref[K + c, :, :]

        for c in range(K):
            if not (3 * K // 4 <= c < K):
                a_rdmas[c][0].wait_recv()
            if not (K // 2 <= c < 3 * K // 4):
                a_rdmas[c][1].wait_recv()
        for b in b_rdmas:
            b.wait_recv()
        for r in xr:
            r.wait_send()
        for ay, az in a_rdmas:
            ay.wait_send()
            az.wait_send()
        for b in b_rdmas:
            b.wait_send()

    return pl.pallas_call(
        body,
        out_shape=jax.ShapeDtypeStruct((m, half), jnp.float32),
        in_specs=[pl.BlockSpec(memory_space=pltpu.VMEM)],
        out_specs=pl.BlockSpec(memory_space=pltpu.VMEM),
        scratch_shapes=[
            pltpu.VMEM((n_x, cm, half), jnp.float32),
            pltpu.SemaphoreType.DMA((n_sem,)),
            pltpu.SemaphoreType.DMA((n_sem,)),
        ],
        compiler_params=pltpu.CompilerParams(collective_id=0),
    )(x)
